# Initial kernel scaffold; baseline (speedup 1.0000x reference)
import jax
import jax.numpy as jnp
from jax.experimental import pallas as pl


def kernel(x_nchw, weight_oihw, gamma, beta, running_mean, running_var):
    raise NotImplementedError("write your pallas kernel here")


# speedup vs baseline: 10.0746x; 10.0746x over previous
import functools

import jax
import jax.numpy as jnp
from jax.experimental import pallas as pl
from jax.experimental.pallas import tpu as pltpu


def _conv_kernel(x_ref, w_ref, shift_ref, o_ref, *, taps, m):
    x2 = x_ref[0]
    patches = jnp.concatenate([x2[s:s + m, :] for s in taps], axis=1)
    acc = jnp.dot(patches, w_ref[...], preferred_element_type=jnp.float32)
    o_ref[0] = jnp.maximum(acc + shift_ref[...], 0.0)


@jax.jit
def _basic_conv2d_opt(x_nchw, weight_oihw, gamma, beta, running_mean,
                      running_var):
    eps = 1e-3
    n, c_in, h, w = x_nchw.shape
    c_out, c_in_w, kh, kw = weight_oihw.shape
    assert c_in == c_in_w
    oh, ow = h, w

    wp = -(-(w + kw - 1) // 8) * 8
    hp = h + kh

    x_nhwc = jnp.transpose(x_nchw, (0, 2, 3, 1)).astype(jnp.bfloat16)
    x_pad = jnp.pad(x_nhwc, ((0, 0), (1, hp - h - 1), (1, wp - w - 1), (0, 0)))
    x_flat = x_pad.reshape(n, hp * wp, c_in)

    scale = gamma.astype(jnp.float32) / jnp.sqrt(
        running_var.astype(jnp.float32) + eps)
    shift = beta.astype(jnp.float32) - running_mean.astype(jnp.float32) * scale
    k_dim = kh * kw * c_in
    w_mat = jnp.transpose(weight_oihw, (2, 3, 1, 0)).reshape(k_dim, c_out)
    w_mat = (w_mat.astype(jnp.float32) * scale[None, :]).astype(jnp.bfloat16)
    shift_row = shift.reshape(1, c_out)

    m = oh * wp
    taps = tuple(i * wp + j for i in range(kh) for j in range(kw))

    out_flat = pl.pallas_call(
        functools.partial(_conv_kernel, taps=taps, m=m),
        out_shape=jax.ShapeDtypeStruct((n, m, c_out), jnp.float32),
        grid_spec=pltpu.PrefetchScalarGridSpec(
            num_scalar_prefetch=0,
            grid=(n,),
            in_specs=[
                pl.BlockSpec((1, hp * wp, c_in), lambda i: (i, 0, 0)),
                pl.BlockSpec((k_dim, c_out), lambda i: (0, 0)),
                pl.BlockSpec((1, c_out), lambda i: (0, 0)),
            ],
            out_specs=pl.BlockSpec((1, m, c_out), lambda i: (i, 0, 0)),
        ),
        compiler_params=pltpu.CompilerParams(
            dimension_semantics=("parallel",),
            vmem_limit_bytes=64 * 1024 * 1024,
        ),
        cost_estimate=pl.CostEstimate(
            flops=2 * n * m * k_dim * c_out,
            transcendentals=0,
            bytes_accessed=n * (hp * wp * c_in * 2 + m * c_out * 4)
            + k_dim * c_out * 2,
        ),
    )(x_flat, w_mat, shift_row)

    out = out_flat.reshape(n, oh, wp, c_out)[:, :, :ow, :]
    return jnp.transpose(out, (0, 3, 1, 2)).astype(jnp.float32)


def kernel(x_nchw, weight_oihw, gamma, beta, running_mean, running_var):
    return _basic_conv2d_opt(x_nchw, weight_oihw, gamma, beta, running_mean,
                             running_var)
